# baseline (speedup 1.0000x reference)
import jax
import jax.numpy as jnp
from jax import lax
from jax.experimental import pallas as pl
from jax.experimental.pallas import tpu as pltpu
from jax.experimental.pallas import tpu_sc as plsc

_VOCAB = 100000
_HID = 768
_BATCH = 1024
_SEQ = 50
_TOTAL = _BATCH * _SEQ
_NW = 32
_BPW = _TOTAL // _NW
_NB = _BPW // _SEQ
_STRIDE = 56
_NVT = 4


def _body(x_hbm, orig_hbm, new_hbm, out_hbm,
          idx_v, idx_safe, blkmax, tree, rows0, rows1, tail0, tail1,
          rowbuf, gsem0, gsem1, wsem0, wsem1):
    wid = lax.axis_index("s") * 2 + lax.axis_index("c")
    base = wid * _BPW
    bat_base = wid * _NB

    pltpu.sync_copy(x_hbm.at[pl.ds(base, _BPW)], idx_v.at[pl.ds(0, _BPW)])

    zeros16 = jnp.zeros((16,), jnp.int32)
    tree[pl.ds(16, 16)] = zeros16
    tail_keep = lax.iota(jnp.int32, 16) < (_SEQ - 3 * 16)

    def clamp(b, carry):
        m = zeros16
        for t in range(_NVT):
            v = idx_v[pl.ds(b * _SEQ + t * 16, 16)]
            safe = jnp.where(v >= _VOCAB, 0, v)
            if t == _NVT - 1:
                safe = jnp.where(tail_keep, safe, 0)
                v = jnp.where(tail_keep, v, 0)
            idx_safe[pl.ds(b * _STRIDE + t * 16, 16)] = safe
            m = jnp.maximum(m, v)
        blkmax[pl.ds(b * 16, 16)] = m
        return carry

    lax.fori_loop(0, _NB, clamp, jnp.int32(0))

    def scan_batch(b):
        m = blkmax[pl.ds(b * 16, 16)]
        for sh in (8, 4, 2, 1):
            tree[pl.ds(0, 16)] = m
            m = jnp.maximum(m, tree[pl.ds(sh, 16)])

        @pl.when(m[0] >= _VOCAB)
        def _():
            def fix_lane(k, c2):
                s = idx_v[pl.ds(b * _SEQ + k, 16)][0]

                @pl.when(s >= _VOCAB)
                def _():
                    pltpu.sync_copy(
                        new_hbm.at[pl.ds(s - (_VOCAB - 1), 1)], rowbuf)
                    pltpu.sync_copy(
                        rowbuf, out_hbm.at[bat_base + b, pl.ds(k, 1)])

                return c2

            lax.fori_loop(0, _SEQ, fix_lane, jnp.int32(0))

    bufs = (rows0, rows1)
    tailbufs = (tail0, tail1)
    gsems = (gsem0, gsem1)
    wsems = (wsem0, wsem1)

    def g_copies(b):
        p = b & 1
        return [
            pltpu.make_async_copy(
                orig_hbm.at[idx_safe.at[pl.ds(b * _STRIDE, 48)]],
                bufs[p].at[pl.ds(0, 48)], gsems[p]),
            pltpu.make_async_copy(
                orig_hbm.at[idx_safe.at[pl.ds(b * _STRIDE + 48, 8)]],
                tailbufs[p], gsems[p]),
        ]

    def g_start(b):
        for c in g_copies(b):
            c.start()

    def g_wait(b):
        for c in g_copies(b):
            c.wait()

    def tail_fix(b):
        p = b & 1

        def cp(c, carry):
            tcol = pl.ds(c * 16, 16)
            bufs[p][48, tcol] = tailbufs[p][0, tcol]
            bufs[p][49, tcol] = tailbufs[p][1, tcol]
            return carry

        lax.fori_loop(0, _HID // 16, cp, jnp.int32(0))

    def w_copy(b):
        p = b & 1
        return pltpu.make_async_copy(
            bufs[p], out_hbm.at[bat_base + b], wsems[p])

    for b in range(_NB):
        if b >= 2:
            w_copy(b - 2).wait()
        g_start(b)
        if b >= 1:
            g_wait(b - 1)
            tail_fix(b - 1)
            w_copy(b - 1).start()
        if b >= 2:
            scan_batch(b - 2)
    g_wait(_NB - 1)
    tail_fix(_NB - 1)
    w_copy(_NB - 1).start()
    w_copy(_NB - 2).wait()
    scan_batch(_NB - 2)
    w_copy(_NB - 1).wait()
    scan_batch(_NB - 1)


_gather = pl.kernel(
    _body,
    out_type=jax.ShapeDtypeStruct((_BATCH, _SEQ, _HID), jnp.float32),
    mesh=plsc.VectorSubcoreMesh(core_axis_name="c", subcore_axis_name="s"),
    scratch_types=[
        pltpu.VMEM((_BPW + 16,), jnp.int32),
        pltpu.VMEM((_NB * _STRIDE + 16,), jnp.int32),
        pltpu.VMEM((_NB * 16,), jnp.int32),
        pltpu.VMEM((32,), jnp.int32),
        pltpu.VMEM((_SEQ, _HID), jnp.float32),
        pltpu.VMEM((_SEQ, _HID), jnp.float32),
        pltpu.VMEM((8, _HID), jnp.float32),
        pltpu.VMEM((8, _HID), jnp.float32),
        pltpu.VMEM((1, _HID), jnp.float32),
        pltpu.SemaphoreType.DMA,
        pltpu.SemaphoreType.DMA,
        pltpu.SemaphoreType.DMA,
        pltpu.SemaphoreType.DMA,
    ],
)


def kernel(x, orig_table, new_table):
    return _gather(x.reshape(-1), orig_table, new_table)


# speedup vs baseline: 2.6195x; 2.6195x over previous
import jax
import jax.numpy as jnp
from jax import lax
from jax.experimental import pallas as pl
from jax.experimental.pallas import tpu as pltpu
from jax.experimental.pallas import tpu_sc as plsc

_VOCAB = 100000
_HID = 768
_BATCH = 1024
_SEQ = 50
_TOTAL = _BATCH * _SEQ
_NW = 32
_BPW = _TOTAL // _NW
_NB = _BPW // _SEQ
_STRIDE = 56
_NVT = 4


def _body(x_hbm, orig_hbm, new_hbm, out_hbm,
          idx_v, idx_safe, blkmax, tree, rows0, rows1,
          rowbuf, gsem0, gsem1, wsem0, wsem1):
    wid = lax.axis_index("s") * 2 + lax.axis_index("c")
    base = wid * _BPW
    bat_base = wid * _NB

    pltpu.sync_copy(x_hbm.at[pl.ds(base, _BPW)], idx_v.at[pl.ds(0, _BPW)])

    zeros16 = jnp.zeros((16,), jnp.int32)
    tree[pl.ds(16, 16)] = zeros16
    tail_keep = lax.iota(jnp.int32, 16) < (_SEQ - 3 * 16)

    def clamp(b, carry):
        m = zeros16
        for t in range(_NVT):
            v = idx_v[pl.ds(b * _SEQ + t * 16, 16)]
            safe = jnp.where(v >= _VOCAB, 0, v)
            if t == _NVT - 1:
                safe = jnp.where(tail_keep, safe, 0)
                v = jnp.where(tail_keep, v, 0)
            idx_safe[pl.ds(b * _STRIDE + t * 16, 16)] = safe
            m = jnp.maximum(m, v)
        blkmax[pl.ds(b * 16, 16)] = m
        return carry

    lax.fori_loop(0, _NB, clamp, jnp.int32(0))

    def scan_batch(b):
        m = blkmax[pl.ds(b * 16, 16)]
        for sh in (8, 4, 2, 1):
            tree[pl.ds(0, 16)] = m
            m = jnp.maximum(m, tree[pl.ds(sh, 16)])

        @pl.when(m[0] >= _VOCAB)
        def _():
            def fix_lane(k, c2):
                s = idx_v[pl.ds(b * _SEQ + k, 16)][0]

                @pl.when(s >= _VOCAB)
                def _():
                    pltpu.sync_copy(
                        new_hbm.at[pl.ds(s - (_VOCAB - 1), 1)], rowbuf)
                    pltpu.sync_copy(
                        rowbuf, out_hbm.at[bat_base + b, pl.ds(k, 1)])

                return c2

            lax.fori_loop(0, _SEQ, fix_lane, jnp.int32(0))

    bufs = (rows0, rows1)
    gsems = (gsem0, gsem1)
    wsems = (wsem0, wsem1)

    def g_copies(b):
        p = b & 1
        tail = idx_safe[pl.ds(b * _STRIDE + 48, 16)]
        i48 = tail[0]
        i49 = tail[1]
        return [
            pltpu.make_async_copy(
                orig_hbm.at[idx_safe.at[pl.ds(b * _STRIDE, 48)]],
                bufs[p].at[pl.ds(0, 48)], gsems[p]),
            pltpu.make_async_copy(
                orig_hbm.at[pl.ds(i48, 1)],
                bufs[p].at[pl.ds(48, 1)], gsems[p]),
            pltpu.make_async_copy(
                orig_hbm.at[pl.ds(i49, 1)],
                bufs[p].at[pl.ds(49, 1)], gsems[p]),
        ]

    def g_start(b):
        for c in g_copies(b):
            c.start()

    def g_wait(b):
        for c in g_copies(b):
            c.wait()

    def w_copy(b):
        p = b & 1
        return pltpu.make_async_copy(
            bufs[p], out_hbm.at[bat_base + b], wsems[p])

    for b in range(_NB):
        if b >= 2:
            w_copy(b - 2).wait()
        g_start(b)
        if b >= 1:
            g_wait(b - 1)
            w_copy(b - 1).start()
        if b >= 2:
            scan_batch(b - 2)
    g_wait(_NB - 1)
    w_copy(_NB - 1).start()
    w_copy(_NB - 2).wait()
    scan_batch(_NB - 2)
    w_copy(_NB - 1).wait()
    scan_batch(_NB - 1)


_gather = pl.kernel(
    _body,
    out_type=jax.ShapeDtypeStruct((_BATCH, _SEQ, _HID), jnp.float32),
    mesh=plsc.VectorSubcoreMesh(core_axis_name="c", subcore_axis_name="s"),
    scratch_types=[
        pltpu.VMEM((_BPW + 16,), jnp.int32),
        pltpu.VMEM((_NB * _STRIDE + 16,), jnp.int32),
        pltpu.VMEM((_NB * 16,), jnp.int32),
        pltpu.VMEM((32,), jnp.int32),
        pltpu.VMEM((_SEQ, _HID), jnp.float32),
        pltpu.VMEM((_SEQ, _HID), jnp.float32),
        pltpu.VMEM((1, _HID), jnp.float32),
        pltpu.SemaphoreType.DMA,
        pltpu.SemaphoreType.DMA,
        pltpu.SemaphoreType.DMA,
        pltpu.SemaphoreType.DMA,
    ],
)


def kernel(x, orig_table, new_table):
    return _gather(x.reshape(-1), orig_table, new_table)


# speedup vs baseline: 2.6197x; 1.0001x over previous
import jax
import jax.numpy as jnp
from jax import lax
from jax.experimental import pallas as pl
from jax.experimental.pallas import tpu as pltpu
from jax.experimental.pallas import tpu_sc as plsc

_VOCAB = 100000
_HID = 768
_BATCH = 1024
_SEQ = 50
_TOTAL = _BATCH * _SEQ
_NW = 32
_BPW = _TOTAL // _NW
_NB = _BPW // _SEQ
_STRIDE = 56
_NVT = 4


def _body(x_hbm, orig_hbm, new_hbm, out_hbm,
          idx_v, idx_safe, blkmax, tree, rows0, rows1,
          rowbuf, gsem0, gsem1, wsem0, wsem1):
    wid = lax.axis_index("s") * 2 + lax.axis_index("c")
    base = wid * _BPW
    bat_base = wid * _NB

    pltpu.sync_copy(x_hbm.at[pl.ds(base, _BPW)], idx_v.at[pl.ds(0, _BPW)])

    zeros16 = jnp.zeros((16,), jnp.int32)
    tree[pl.ds(16, 16)] = zeros16
    tail_keep = lax.iota(jnp.int32, 16) < (_SEQ - 3 * 16)

    def clamp(b, carry):
        m = zeros16
        for t in range(_NVT):
            v = idx_v[pl.ds(b * _SEQ + t * 16, 16)]
            safe = jnp.where(v >= _VOCAB, 0, v)
            if t == _NVT - 1:
                safe = jnp.where(tail_keep, safe, 0)
                v = jnp.where(tail_keep, v, 0)
            idx_safe[pl.ds(b * _STRIDE + t * 16, 16)] = safe
            m = jnp.maximum(m, v)
        blkmax[pl.ds(b * 16, 16)] = m
        return carry

    lax.fori_loop(0, _NB, clamp, jnp.int32(0))

    def scan_batch(b):
        m = blkmax[pl.ds(b * 16, 16)]
        for sh in (8, 4, 2, 1):
            tree[pl.ds(0, 16)] = m
            m = jnp.maximum(m, tree[pl.ds(sh, 16)])

        @pl.when(m[0] >= _VOCAB)
        def _():
            def fix_lane(k, c2):
                s = idx_v[pl.ds(b * _SEQ + k, 16)][0]

                @pl.when(s >= _VOCAB)
                def _():
                    pltpu.sync_copy(
                        new_hbm.at[pl.ds(s - (_VOCAB - 1), 1)], rowbuf)
                    pltpu.sync_copy(
                        rowbuf, out_hbm.at[bat_base + b, pl.ds(k, 1)])

                return c2

            lax.fori_loop(0, _SEQ, fix_lane, jnp.int32(0))

    bufs = (rows0, rows1)
    gsems = (gsem0, gsem1)
    wsems = (wsem0, wsem1)

    def g_copies(b):
        p = b & 1
        tail = idx_safe[pl.ds(b * _STRIDE + 48, 16)]
        i48 = tail[0]
        i49 = tail[1]
        return [
            pltpu.make_async_copy(
                orig_hbm.at[idx_safe.at[pl.ds(b * _STRIDE, 48)]],
                bufs[p].at[pl.ds(0, 48)], gsems[p]),
            pltpu.make_async_copy(
                orig_hbm.at[pl.ds(i48, 1)],
                bufs[p].at[pl.ds(48, 1)], gsems[p]),
            pltpu.make_async_copy(
                orig_hbm.at[pl.ds(i49, 1)],
                bufs[p].at[pl.ds(49, 1)], gsems[p]),
        ]

    def g_start(b):
        for c in g_copies(b):
            c.start()

    def g_wait(b):
        for c in g_copies(b):
            c.wait()

    def w_copy(b):
        p = b & 1
        return pltpu.make_async_copy(
            bufs[p], out_hbm.at[bat_base + b], wsems[p])

    for b in range(_NB):
        if b >= 2:
            w_copy(b - 2).wait()
        g_start(b)
        if b >= 1:
            g_wait(b - 1)
            w_copy(b - 1).start()
        if b >= 2:
            scan_batch(b - 2)
    g_wait(_NB - 1)
    w_copy(_NB - 1).start()
    w_copy(_NB - 2).wait()
    scan_batch(_NB - 2)
    w_copy(_NB - 1).wait()
    scan_batch(_NB - 1)


_gather = pl.kernel(
    _body,
    out_type=(),
    mesh=plsc.VectorSubcoreMesh(core_axis_name="c", subcore_axis_name="s"),
    scratch_types=[
        pltpu.VMEM((_BPW + 16,), jnp.int32),
        pltpu.VMEM((_NB * _STRIDE + 16,), jnp.int32),
        pltpu.VMEM((_NB * 16,), jnp.int32),
        pltpu.VMEM((32,), jnp.int32),
        pltpu.VMEM((_SEQ, _HID), jnp.float32),
        pltpu.VMEM((_SEQ, _HID), jnp.float32),
        pltpu.VMEM((1, _HID), jnp.float32),
        pltpu.SemaphoreType.DMA,
        pltpu.SemaphoreType.DMA,
        pltpu.SemaphoreType.DMA,
        pltpu.SemaphoreType.DMA,
    ],
)


def kernel(x, orig_table, new_table):
    out_ref = jax.empty_ref(
        jax.ShapeDtypeStruct((_BATCH, _SEQ, _HID), jnp.float32))
    _gather(x.reshape(-1), orig_table, new_table, out_ref)
    out = out_ref[...]
    jax.free_ref(out_ref)
    return out


# speedup vs baseline: 2.6199x; 1.0001x over previous
import jax
import jax.numpy as jnp
from jax import lax
from jax.experimental import pallas as pl
from jax.experimental.pallas import tpu as pltpu
from jax.experimental.pallas import tpu_sc as plsc

_VOCAB = 100000
_HID = 768
_BATCH = 1024
_SEQ = 50
_TOTAL = _BATCH * _SEQ
_NW = 32
_BPW = _TOTAL // _NW
_NB = _BPW // _SEQ
_STRIDE = 56
_NVT = 4


def _body(x_hbm, orig_hbm, new_hbm, out_hbm,
          idx_v, idx_safe, blkmax, tree, rows0, rows1,
          rowbuf, gsem0, gsem1, wsem0, wsem1):
    wid = lax.axis_index("s") * 2 + lax.axis_index("c")
    base = wid * _BPW
    bat_base = wid * _NB

    pltpu.sync_copy(x_hbm.at[pl.ds(base, _BPW)], idx_v.at[pl.ds(0, _BPW)])

    zeros16 = jnp.zeros((16,), jnp.int32)
    tree[pl.ds(16, 16)] = zeros16
    tail_keep = lax.iota(jnp.int32, 16) < (_SEQ - 3 * 16)

    def clamp(b, carry):
        m = zeros16
        for t in range(_NVT):
            v = idx_v[pl.ds(b * _SEQ + t * 16, 16)]
            safe = jnp.where(v >= _VOCAB, 0, v)
            if t == _NVT - 1:
                safe = jnp.where(tail_keep, safe, 0)
                v = jnp.where(tail_keep, v, 0)
            idx_safe[pl.ds(b * _STRIDE + t * 16, 16)] = safe
            m = jnp.maximum(m, v)
        blkmax[pl.ds(b * 16, 16)] = m
        return carry

    lax.fori_loop(0, _NB, clamp, jnp.int32(0))

    def scan_batch(b):
        m = blkmax[pl.ds(b * 16, 16)]
        for sh in (8, 4, 2, 1):
            tree[pl.ds(0, 16)] = m
            m = jnp.maximum(m, tree[pl.ds(sh, 16)])

        @pl.when(m[0] >= _VOCAB)
        def _():
            def fix_lane(k, c2):
                s = idx_v[pl.ds(b * _SEQ + k, 16)][0]

                @pl.when(s >= _VOCAB)
                def _():
                    pltpu.sync_copy(
                        new_hbm.at[pl.ds(s - (_VOCAB - 1), 1)], rowbuf)
                    pltpu.sync_copy(
                        rowbuf, out_hbm.at[bat_base + b, pl.ds(k, 1)])

                return c2

            lax.fori_loop(0, _SEQ, fix_lane, jnp.int32(0))

    bufs = (rows0, rows1)
    gsems = (gsem0, gsem1)
    wsems = (wsem0, wsem1)

    def g_copies(b):
        p = b & 1
        tail = idx_safe[pl.ds(b * _STRIDE + 48, 16)]
        i48 = tail[0]
        i49 = tail[1]
        return [
            pltpu.make_async_copy(
                orig_hbm.at[idx_safe.at[pl.ds(b * _STRIDE, 48)]],
                bufs[p].at[pl.ds(0, 48)], gsems[p]),
            pltpu.make_async_copy(
                orig_hbm.at[pl.ds(i48, 1)],
                bufs[p].at[pl.ds(48, 1)], gsems[p]),
            pltpu.make_async_copy(
                orig_hbm.at[pl.ds(i49, 1)],
                bufs[p].at[pl.ds(49, 1)], gsems[p]),
        ]

    def g_start(b):
        for c in g_copies(b):
            c.start()

    def g_wait(b):
        for c in g_copies(b):
            c.wait()

    def w_copy(b):
        p = b & 1
        return pltpu.make_async_copy(
            bufs[p], out_hbm.at[bat_base + b], wsems[p])

    for b in range(_NB):
        if b >= 2:
            w_copy(b - 2).wait()
        g_start(b)
        if b >= 1:
            g_wait(b - 1)
            w_copy(b - 1).start()
        if b >= 2:
            scan_batch(b - 2)
    g_wait(_NB - 1)
    w_copy(_NB - 1).start()
    w_copy(_NB - 2).wait()
    scan_batch(_NB - 2)
    w_copy(_NB - 1).wait()
    scan_batch(_NB - 1)


_gather = pl.kernel(
    _body,
    out_type=jax.ShapeDtypeStruct((_BATCH, _SEQ, _HID), jnp.float32),
    mesh=plsc.VectorSubcoreMesh(core_axis_name="c", subcore_axis_name="s"),
    scratch_types=[
        pltpu.VMEM((_BPW + 16,), jnp.int32),
        pltpu.VMEM((_NB * _STRIDE + 16,), jnp.int32),
        pltpu.VMEM((_NB * 16,), jnp.int32),
        pltpu.VMEM((32,), jnp.int32),
        pltpu.VMEM((_SEQ, _HID), jnp.float32),
        pltpu.VMEM((_SEQ, _HID), jnp.float32),
        pltpu.VMEM((1, _HID), jnp.float32),
        pltpu.SemaphoreType.DMA,
        pltpu.SemaphoreType.DMA,
        pltpu.SemaphoreType.DMA,
        pltpu.SemaphoreType.DMA,
    ],
)


def kernel(x, orig_table, new_table):
    return _gather(x.reshape(-1), orig_table, new_table)
